# baseline (speedup 1.0000x reference)
import dataclasses
import functools

import jax
import jax.numpy as jnp
from jax import lax
from jax.experimental import pallas as pl
from jax.experimental.pallas import tpu as pltpu
from jax.experimental.pallas import tpu_sc as plsc

NUM_FIELDS = 26
VOCAB = 100000
EMB_DIM = 32
NUM_NUM = 13
HIDDEN = 128
BATCH = 16384

SC_CORES = 2
SC_SUBCORES = 16
GATHER_CHUNK = 128
MLP_BLOCK = 512


def _sc_gather(table_wide, g_idx, r_idx):
    num_indices = g_idx.shape[0]
    n_workers = SC_CORES * SC_SUBCORES
    rows_per_worker = num_indices // n_workers
    C = GATHER_CHUNK
    n_chunks = rows_per_worker // C
    assert rows_per_worker % C == 0
    mesh = plsc.VectorSubcoreMesh(core_axis_name="c", subcore_axis_name="s")
    cp = pltpu.CompilerParams()
    if "needs_layout_passes" in pltpu.CompilerParams.__dataclass_fields__:
        cp = dataclasses.replace(cp, needs_layout_passes=False)

    @functools.partial(
        pl.kernel,
        mesh=mesh,
        compiler_params=cp,
        out_type=jax.ShapeDtypeStruct((num_indices, EMB_DIM), jnp.float32),
        scratch_types=[
            pltpu.VMEM((rows_per_worker,), jnp.int32),
            pltpu.VMEM((rows_per_worker,), jnp.int32),
            pltpu.VMEM((rows_per_worker,), jnp.int32),
            pltpu.VMEM((C, 128), jnp.float32),
            pltpu.VMEM((C, 128), jnp.float32),
            pltpu.VMEM((C, EMB_DIM), jnp.float32),
            pltpu.VMEM((C, EMB_DIM), jnp.float32),
            pltpu.SemaphoreType.DMA,
            pltpu.SemaphoreType.DMA,
            pltpu.SemaphoreType.DMA,
            pltpu.SemaphoreType.DMA,
        ],
    )
    def gather_kernel(table_hbm, g_hbm, r_hbm, out_hbm, g_all, r_all, c_all,
                      buf0, buf1, rows0, rows1, sem0, sem1, semw0, semw1):
        wid = lax.axis_index("s") * SC_CORES + lax.axis_index("c")
        base = wid * rows_per_worker
        lane = lax.broadcasted_iota(jnp.int32, (16,), 0)
        H = C // 2

        pltpu.sync_copy(g_hbm.at[pl.ds(base, rows_per_worker)], g_all)
        pltpu.sync_copy(r_hbm.at[pl.ds(base, rows_per_worker)], r_all)

        @pl.loop(0, rows_per_worker, step=16)
        def _(i):
            c_all[pl.ds(i, 16)] = r_all[pl.ds(i, 16)] * 32

        def start_gather(k, buf, sem):
            pltpu.async_copy(
                table_hbm.at[g_all.at[pl.ds(k * C, H)]],
                buf.at[pl.ds(0, H)], sem)
            pltpu.async_copy(
                table_hbm.at[g_all.at[pl.ds(k * C + H, H)]],
                buf.at[pl.ds(H, H)], sem)

        def wait_gather(buf, sem):
            pltpu.make_async_copy(
                table_hbm.at[pl.ds(0, H)], buf.at[pl.ds(0, H)], sem).wait()
            pltpu.make_async_copy(
                table_hbm.at[pl.ds(0, H)], buf.at[pl.ds(H, H)], sem).wait()

        def extract_and_flush(buf, rows_v, semw, k, t):
            @pl.when(t > 0)
            def _():
                pltpu.make_async_copy(
                    rows_v, out_hbm.at[pl.ds(0, C)], semw).wait()

            @pl.loop(0, C, step=16)
            def _(i):
                row16 = i + lane
                col_base = c_all[pl.ds(k * C + i, 16)]
                for j in range(EMB_DIM):
                    vals = plsc.load_gather(buf, [row16, col_base + j])
                    plsc.store_scatter(rows_v, [row16, lane * 0 + j], vals)

            pltpu.async_copy(rows_v, out_hbm.at[pl.ds(base + k * C, C)], semw)

        n2 = n_chunks // 2
        start_gather(0, buf0, sem0)

        @pl.loop(0, n2)
        def _(t):
            k0 = 2 * t
            start_gather(k0 + 1, buf1, sem1)
            wait_gather(buf0, sem0)
            extract_and_flush(buf0, rows0, semw0, k0, t)

            @pl.when(t + 1 < n2)
            def _():
                start_gather(k0 + 2, buf0, sem0)

            wait_gather(buf1, sem1)
            extract_and_flush(buf1, rows1, semw1, k0 + 1, t)

        pltpu.make_async_copy(rows0, out_hbm.at[pl.ds(0, C)], semw0).wait()
        pltpu.make_async_copy(rows1, out_hbm.at[pl.ds(0, C)], semw1).wait()

    return gather_kernel(table_wide, g_idx, r_idx)


def _compact_kernel(x0_ref, x1_ref, x2_ref, x3_ref, o_ref):
    o_ref[...] = jnp.concatenate(
        [x0_ref[...], x1_ref[...], x2_ref[...], x3_ref[...]], axis=1)


QUARTER = NUM_FIELDS * VOCAB // 4


def _tc_compact(tables_flat):
    blk = 1000
    nb = QUARTER // blk
    grid = (nb,)
    specs = [
        pl.BlockSpec((blk, EMB_DIM),
                     functools.partial(lambda a, i: (a * nb + i, 0), a))
        for a in range(4)
    ]
    return pl.pallas_call(
        _compact_kernel,
        grid=grid,
        in_specs=specs,
        out_specs=pl.BlockSpec((blk, 4 * EMB_DIM), lambda i: (i, 0)),
        out_shape=jax.ShapeDtypeStruct((QUARTER, 4 * EMB_DIM), jnp.float32),
    )(tables_flat, tables_flat, tables_flat, tables_flat)


def _mlp_kernel(emb_ref, xn_ref, w1e_ref, w1n_ref, b1_ref, w2_ref, b2_ref,
                w3_ref, b3_ref, o_ref):
    x = jnp.concatenate([emb_ref[:, f, :] for f in range(NUM_FIELDS)], axis=1)
    h = jnp.dot(x, w1e_ref[...], preferred_element_type=jnp.float32)
    h = h + jnp.dot(xn_ref[...], w1n_ref[...], preferred_element_type=jnp.float32)
    h = jax.nn.relu(h + b1_ref[...])
    h = jax.nn.relu(jnp.dot(h, w2_ref[...], preferred_element_type=jnp.float32)
                    + b2_ref[...])
    o_ref[...] = (jnp.dot(h, w3_ref[...], preferred_element_type=jnp.float32)
                  + b3_ref[...])


def _tc_mlp(emb3, x_numerical, W1, b1, W2, b2, W3, b3):
    batch = emb3.shape[0]
    emb_width = NUM_FIELDS * EMB_DIM
    W1e = W1[:emb_width]
    W1n = W1[emb_width:]
    grid = (batch // MLP_BLOCK,)
    return pl.pallas_call(
        _mlp_kernel,
        grid=grid,
        in_specs=[
            pl.BlockSpec((MLP_BLOCK, NUM_FIELDS, EMB_DIM), lambda i: (i, 0, 0)),
            pl.BlockSpec((MLP_BLOCK, NUM_NUM), lambda i: (i, 0)),
            pl.BlockSpec((emb_width, HIDDEN), lambda i: (0, 0)),
            pl.BlockSpec((NUM_NUM, HIDDEN), lambda i: (0, 0)),
            pl.BlockSpec((1, HIDDEN), lambda i: (0, 0)),
            pl.BlockSpec((HIDDEN, HIDDEN // 2), lambda i: (0, 0)),
            pl.BlockSpec((1, HIDDEN // 2), lambda i: (0, 0)),
            pl.BlockSpec((HIDDEN // 2, 1), lambda i: (0, 0)),
            pl.BlockSpec((1, 1), lambda i: (0, 0)),
        ],
        out_specs=pl.BlockSpec((MLP_BLOCK, 1), lambda i: (i, 0)),
        out_shape=jax.ShapeDtypeStruct((batch, 1), jnp.float32),
    )(emb3, x_numerical, W1e, W1n, b1.reshape(1, HIDDEN), W2,
      b2.reshape(1, HIDDEN // 2), W3, b3.reshape(1, 1))


def kernel(x_categorical, x_numerical, tables, W1, b1, W2, b2, W3, b3):
    offsets = (jnp.arange(NUM_FIELDS, dtype=jnp.int32) * VOCAB)[None, :]
    flat_idx = (x_categorical + offsets).reshape(BATCH * NUM_FIELDS)
    g_idx = flat_idx % QUARTER
    r_idx = flat_idx // QUARTER
    table_wide = _tc_compact(tables.reshape(NUM_FIELDS * VOCAB, EMB_DIM))
    emb = _sc_gather(table_wide, g_idx, r_idx)
    emb3 = emb.reshape(BATCH, NUM_FIELDS, EMB_DIM)
    return _tc_mlp(emb3, x_numerical, W1, b1, W2, b2, W3, b3)


# speedup vs baseline: 1.1267x; 1.1267x over previous
import dataclasses
import functools

import jax
import jax.numpy as jnp
from jax import lax
from jax.experimental import pallas as pl
from jax.experimental.pallas import tpu as pltpu
from jax.experimental.pallas import tpu_sc as plsc

NUM_FIELDS = 26
VOCAB = 100000
EMB_DIM = 32
NUM_NUM = 13
HIDDEN = 128
BATCH = 16384

SC_CORES = 2
SC_SUBCORES = 16
GATHER_CHUNK = 128
MLP_BLOCK = 1024


def _sc_gather(table_wide, g_idx, r_idx):
    num_indices = g_idx.shape[0]
    n_workers = SC_CORES * SC_SUBCORES
    rows_per_worker = num_indices // n_workers
    C = GATHER_CHUNK
    n_chunks = rows_per_worker // C
    assert rows_per_worker % C == 0
    mesh = plsc.VectorSubcoreMesh(core_axis_name="c", subcore_axis_name="s")
    cp = pltpu.CompilerParams()
    if "needs_layout_passes" in pltpu.CompilerParams.__dataclass_fields__:
        cp = dataclasses.replace(cp, needs_layout_passes=False)

    @functools.partial(
        pl.kernel,
        mesh=mesh,
        compiler_params=cp,
        out_type=jax.ShapeDtypeStruct((num_indices, EMB_DIM), jnp.float32),
        scratch_types=[
            pltpu.VMEM((rows_per_worker,), jnp.int32),
            pltpu.VMEM((rows_per_worker,), jnp.int32),
            pltpu.VMEM((rows_per_worker,), jnp.int32),
            pltpu.VMEM((C, 128), jnp.float32),
            pltpu.VMEM((C, 128), jnp.float32),
            pltpu.VMEM((C, EMB_DIM), jnp.float32),
            pltpu.VMEM((C, EMB_DIM), jnp.float32),
            pltpu.SemaphoreType.DMA,
            pltpu.SemaphoreType.DMA,
            pltpu.SemaphoreType.DMA,
            pltpu.SemaphoreType.DMA,
        ],
    )
    def gather_kernel(table_hbm, g_hbm, r_hbm, out_hbm, g_all, r_all, c_all,
                      buf0, buf1, rows0, rows1, sem0, sem1, semw0, semw1):
        wid = lax.axis_index("s") * SC_CORES + lax.axis_index("c")
        base = wid * rows_per_worker
        lane = lax.broadcasted_iota(jnp.int32, (16,), 0)
        H = C // 2

        pltpu.sync_copy(g_hbm.at[pl.ds(base, rows_per_worker)], g_all)
        pltpu.sync_copy(r_hbm.at[pl.ds(base, rows_per_worker)], r_all)

        @pl.loop(0, rows_per_worker, step=16)
        def _(i):
            c_all[pl.ds(i, 16)] = r_all[pl.ds(i, 16)] * 32

        def start_gather(k, buf, sem):
            pltpu.async_copy(
                table_hbm.at[g_all.at[pl.ds(k * C, H)]],
                buf.at[pl.ds(0, H)], sem)
            pltpu.async_copy(
                table_hbm.at[g_all.at[pl.ds(k * C + H, H)]],
                buf.at[pl.ds(H, H)], sem)

        def wait_gather(buf, sem):
            pltpu.make_async_copy(
                table_hbm.at[pl.ds(0, H)], buf.at[pl.ds(0, H)], sem).wait()
            pltpu.make_async_copy(
                table_hbm.at[pl.ds(0, H)], buf.at[pl.ds(H, H)], sem).wait()

        def extract_and_flush(buf, rows_v, semw, k, t):
            @pl.when(t > 0)
            def _():
                pltpu.make_async_copy(
                    rows_v, out_hbm.at[pl.ds(0, C)], semw).wait()

            @pl.loop(0, C, step=16)
            def _(i):
                row16 = i + lane
                col_base = c_all[pl.ds(k * C + i, 16)]
                for j in range(EMB_DIM):
                    vals = plsc.load_gather(buf, [row16, col_base + j])
                    plsc.store_scatter(rows_v, [row16, lane * 0 + j], vals)

            pltpu.async_copy(rows_v, out_hbm.at[pl.ds(base + k * C, C)], semw)

        n2 = n_chunks // 2
        start_gather(0, buf0, sem0)

        @pl.loop(0, n2)
        def _(t):
            k0 = 2 * t
            start_gather(k0 + 1, buf1, sem1)
            wait_gather(buf0, sem0)
            extract_and_flush(buf0, rows0, semw0, k0, t)

            @pl.when(t + 1 < n2)
            def _():
                start_gather(k0 + 2, buf0, sem0)

            wait_gather(buf1, sem1)
            extract_and_flush(buf1, rows1, semw1, k0 + 1, t)

        pltpu.make_async_copy(rows0, out_hbm.at[pl.ds(0, C)], semw0).wait()
        pltpu.make_async_copy(rows1, out_hbm.at[pl.ds(0, C)], semw1).wait()

    return gather_kernel(table_wide, g_idx, r_idx)


def _compact_kernel(x0_ref, x1_ref, x2_ref, x3_ref, o_ref):
    o_ref[...] = jnp.concatenate(
        [x0_ref[...], x1_ref[...], x2_ref[...], x3_ref[...]], axis=1)


QUARTER = NUM_FIELDS * VOCAB // 4


def _tc_compact(tables_flat):
    blk = 1000
    nb = QUARTER // blk
    grid = (nb,)
    specs = [
        pl.BlockSpec((blk, EMB_DIM),
                     functools.partial(lambda a, i: (a * nb + i, 0), a))
        for a in range(4)
    ]
    return pl.pallas_call(
        _compact_kernel,
        grid=grid,
        in_specs=specs,
        out_specs=pl.BlockSpec((blk, 4 * EMB_DIM), lambda i: (i, 0)),
        out_shape=jax.ShapeDtypeStruct((QUARTER, 4 * EMB_DIM), jnp.float32),
    )(tables_flat, tables_flat, tables_flat, tables_flat)


def _mlp_kernel(emb_ref, xn_ref, w1e_ref, w1n_ref, b1_ref, w2_ref, b2_ref,
                w3_ref, b3_ref, o_ref):
    h = jnp.dot(emb_ref[...], w1e_ref[...], preferred_element_type=jnp.float32)
    h = h + jnp.dot(xn_ref[...], w1n_ref[...], preferred_element_type=jnp.float32)
    h = jax.nn.relu(h + b1_ref[...])
    h = jax.nn.relu(jnp.dot(h, w2_ref[...], preferred_element_type=jnp.float32)
                    + b2_ref[...])
    o_ref[...] = (jnp.dot(h, w3_ref[...], preferred_element_type=jnp.float32)
                  + b3_ref[...])


def _tc_mlp(emb_flat, x_numerical, W1, b1, W2, b2, W3, b3):
    batch = emb_flat.shape[0]
    emb_width = NUM_FIELDS * EMB_DIM
    W1e = W1[:emb_width]
    W1n = W1[emb_width:]
    grid = (batch // MLP_BLOCK,)
    return pl.pallas_call(
        _mlp_kernel,
        grid=grid,
        in_specs=[
            pl.BlockSpec((MLP_BLOCK, emb_width), lambda i: (i, 0)),
            pl.BlockSpec((MLP_BLOCK, NUM_NUM), lambda i: (i, 0)),
            pl.BlockSpec((emb_width, HIDDEN), lambda i: (0, 0)),
            pl.BlockSpec((NUM_NUM, HIDDEN), lambda i: (0, 0)),
            pl.BlockSpec((1, HIDDEN), lambda i: (0, 0)),
            pl.BlockSpec((HIDDEN, HIDDEN // 2), lambda i: (0, 0)),
            pl.BlockSpec((1, HIDDEN // 2), lambda i: (0, 0)),
            pl.BlockSpec((HIDDEN // 2, 1), lambda i: (0, 0)),
            pl.BlockSpec((1, 1), lambda i: (0, 0)),
        ],
        out_specs=pl.BlockSpec((MLP_BLOCK, 1), lambda i: (i, 0)),
        out_shape=jax.ShapeDtypeStruct((batch, 1), jnp.float32),
    )(emb_flat, x_numerical, W1e, W1n, b1.reshape(1, HIDDEN), W2,
      b2.reshape(1, HIDDEN // 2), W3, b3.reshape(1, 1))


def kernel(x_categorical, x_numerical, tables, W1, b1, W2, b2, W3, b3):
    offsets = (jnp.arange(NUM_FIELDS, dtype=jnp.int32) * VOCAB)[None, :]
    flat_idx = (x_categorical + offsets).reshape(BATCH * NUM_FIELDS)
    g_idx = flat_idx % QUARTER
    r_idx = flat_idx // QUARTER
    table_wide = _tc_compact(tables.reshape(NUM_FIELDS * VOCAB, EMB_DIM))
    emb = _sc_gather(table_wide, g_idx, r_idx)
    emb_flat = emb.reshape(BATCH, NUM_FIELDS * EMB_DIM)
    return _tc_mlp(emb_flat, x_numerical, W1, b1, W2, b2, W3, b3)
